# baseline (speedup 1.0000x reference)
import functools

import jax
import jax.numpy as jnp
from jax import lax
from jax.experimental import pallas as pl
from jax.experimental.pallas import tpu as pltpu
from jax.experimental.pallas import tpu_sc as plsc

VOCAB_SIZE = 100000
EMBED_DIM = 64
SEQ = 4096
LEN = 50
NUM_CORES = 2
NUM_SUBCORES = 16
NUM_WORKERS = NUM_CORES * NUM_SUBCORES
SEQ_PER_WORKER = SEQ // NUM_WORKERS
SEQ_PER_STEP = 16
NSTEP = SEQ_PER_WORKER // SEQ_PER_STEP
NBUF = 2

_mesh = plsc.VectorSubcoreMesh(core_axis_name="c", subcore_axis_name="s")


@functools.partial(
    pl.kernel,
    out_type=jax.ShapeDtypeStruct((SEQ, LEN, EMBED_DIM), jnp.float32),
    mesh=_mesh,
    scratch_types=[
        pltpu.VMEM((SEQ_PER_WORKER, LEN), jnp.int32),
        pltpu.VMEM((NBUF, SEQ_PER_STEP, LEN, EMBED_DIM), jnp.float32),
        pltpu.SemaphoreType.DMA,
        pltpu.SemaphoreType.DMA,
    ],
    compiler_params=pltpu.CompilerParams(use_tc_tiling_on_sc=False),
)
def _embed_sc(idx_hbm, table_hbm, out_hbm, idx_v, rows_v, gsem, wsem):
    wid = lax.axis_index("s") * NUM_CORES + lax.axis_index("c")
    seq_base = wid * SEQ_PER_WORKER
    pltpu.sync_copy(idx_hbm.at[pl.ds(seq_base, SEQ_PER_WORKER)], idx_v)

    def out_slice(j):
        return out_hbm.at[pl.ds(seq_base + j * SEQ_PER_STEP, SEQ_PER_STEP)]

    def start_gathers(b, j):
        def one(s, c):
            pltpu.async_copy(table_hbm.at[idx_v.at[j * SEQ_PER_STEP + s]],
                             rows_v.at[b, s], gsem)
            return c
        lax.fori_loop(0, SEQ_PER_STEP, one, 0)

    def wait_gathers(b, j):
        pltpu.make_async_copy(out_slice(j), rows_v.at[b], gsem).wait()

    def issue_write(b, j):
        pltpu.async_copy(rows_v.at[b], out_slice(j), wsem)

    def wait_write(b, j):
        pltpu.make_async_copy(rows_v.at[b], out_slice(j), wsem).wait()

    start_gathers(0, 0)
    start_gathers(1, 1)
    wait_gathers(0, 0)
    issue_write(0, 0)

    def body(i, carry):
        for b in range(NBUF):
            j = 1 + i * NBUF + b
            bj = (b + 1) % NBUF
            bp = b
            wait_gathers(bj, j)
            issue_write(bj, j)
            wait_write(bp, j - 1)
            start_gathers(bp, j - 1 + NBUF)
        return carry

    lax.fori_loop(0, (NSTEP - NBUF) // NBUF, body, 0)

    for j in range(NSTEP - NBUF + 1, NSTEP):
        wait_gathers(j % NBUF, j)
        issue_write(j % NBUF, j)
        wait_write((j - 1) % NBUF, j - 1)
    wait_write((NSTEP - 1) % NBUF, NSTEP - 1)


def kernel(input_ids, table):
    embeds = _embed_sc(input_ids.astype(jnp.int32), table)
    return (embeds, embeds, embeds)
